# baseline (speedup 1.0000x reference)
import jax
import jax.numpy as jnp
from jax.experimental import pallas as pl
from jax.experimental.pallas import tpu as pltpu

_HOP = 320
_CONTRACT_LAST = (((1,), (1,)), ((), ()))
_CHUNK = 400


def _vq_body(frames_hbm, W1_hbm, b1_hbm, W2_hbm, b2_hbm, cb_hbm, out_ref,
             f_ref, W1_ref, b1_ref, W2_ref, b2_ref, cb_ref, cb2_ref, sems):
    mt = f_ref.shape[0]
    n_f = mt // _CHUNK
    c_W1 = pltpu.make_async_copy(W1_hbm, W1_ref, sems.at[0])
    c_b1 = pltpu.make_async_copy(b1_hbm, b1_ref, sems.at[1])
    c_W2 = pltpu.make_async_copy(W2_hbm, W2_ref, sems.at[2])
    c_b2 = pltpu.make_async_copy(b2_hbm, b2_ref, sems.at[3])
    c_cb = pltpu.make_async_copy(cb_hbm, cb_ref, sems.at[4])
    c_f = [
        pltpu.make_async_copy(
            frames_hbm.at[pl.ds(j * _CHUNK, _CHUNK), :],
            f_ref.at[pl.ds(j * _CHUNK, _CHUNK), :], sems.at[5 + j])
        for j in range(n_f)
    ]
    c_W1.start()
    c_b1.start()
    c_f[0].start()
    c_W2.start()
    c_b2.start()
    for j in range(1, n_f):
        c_f[j].start()
    c_cb.start()

    c_W1.wait()
    c_b1.wait()
    W1 = W1_ref[...].astype(jnp.bfloat16)
    b1 = b1_ref[...]

    cs, z2s = [], []
    for j in range(n_f):
        c_f[j].wait()
        f = f_ref[pl.ds(j * _CHUNK, _CHUNK), :]
        h = jnp.tanh(
            jnp.dot(f, W1, preferred_element_type=jnp.float32) + b1)
        if j == 0:
            c_W2.wait()
            c_b2.wait()
        c = jnp.tanh(
            jnp.dot(h, W2_ref[...], preferred_element_type=jnp.float32)
            + b2_ref[...])
        cs.append(c)
        z2s.append(jnp.sum(c * c, axis=1, keepdims=True))

    c_cb.wait()
    cb0 = cb_ref[...]
    cb2_ref[...] = jnp.sum(cb0 * cb0, axis=1, keepdims=True).T
    cb2 = cb2_ref[...]
    for j in range(n_f):
        cross = jax.lax.dot_general(cs[j], cb0, _CONTRACT_LAST,
                                    preferred_element_type=jnp.float32)
        s = z2s[j] - 2.0 * cross + cb2
        out_ref[0, 0, pl.ds(j * _CHUNK, _CHUNK)] = jnp.argmin(
            s, axis=1).astype(jnp.int32)


def kernel(x, W1, b1, W2, b2, codebook):
    B = x.shape[0]
    if x.ndim == 3 and x.shape[-1] == 1:
        x = x[..., 0]
    T = x.shape[1] // _HOP
    M = B * T
    G, K, Dg = codebook.shape
    D = W2.shape[1]
    frames = x[:, : T * _HOP].reshape(M, _HOP).astype(jnp.bfloat16)
    n_f = M // _CHUNK

    out = pl.pallas_call(
        _vq_body,
        in_specs=[pl.BlockSpec(memory_space=pl.ANY)] * 6,
        out_shape=jax.ShapeDtypeStruct((1, 1, M), jnp.int32),
        out_specs=pl.BlockSpec((1, 1, M), lambda: (0, 0, 0)),
        scratch_shapes=[
            pltpu.VMEM((M, _HOP), jnp.bfloat16),
            pltpu.VMEM((_HOP, D), jnp.float32),
            pltpu.VMEM((1, D), jnp.float32),
            pltpu.VMEM((D, D), jnp.float32),
            pltpu.VMEM((1, D), jnp.float32),
            pltpu.VMEM((K, Dg), jnp.float32),
            pltpu.VMEM((1, K), jnp.float32),
            pltpu.SemaphoreType.DMA((5 + n_f,)),
        ],
    )(frames, W1, b1[None], W2, b2[None], codebook[0])
    return out.reshape(B, T, G).astype(jnp.int32)
